# baseline (speedup 1.0000x reference)
import functools

import jax
import jax.numpy as jnp
from jax.experimental import pallas as pl
from jax.experimental.pallas import tpu as pltpu

N = 10000
E = 160000
NUM_RELS = 8
NUM_BASES = 4
HID = 512
IN_DIM = 3072
OUT_DIM = 64

MBLK = 1000


def _fnn_body(h_ref, w_ref, b_ref, o_ref):
    acc = jnp.dot(h_ref[...], w_ref[...], preferred_element_type=jnp.float32)
    o_ref[...] = jnp.maximum(acc + b_ref[...], 0.0)


def _fnn(h, W_fnn, b_fnn):
    m = h.shape[0]
    grid = (m // MBLK,)
    return pl.pallas_call(
        _fnn_body,
        grid=grid,
        in_specs=[
            pl.BlockSpec((MBLK, IN_DIM), lambda i: (i, 0)),
            pl.BlockSpec((IN_DIM, 256), lambda i: (0, 0)),
            pl.BlockSpec((1, 256), lambda i: (0, 0)),
        ],
        out_specs=pl.BlockSpec((MBLK, 256), lambda i: (i, 0)),
        out_shape=jax.ShapeDtypeStruct((m, 256), jnp.float32),
    )(h, W_fnn, b_fnn.reshape(1, 256))


def _rel_mm_body(x_ref, w_ref, o_ref):
    o_ref[0] = jnp.dot(x_ref[...], w_ref[0], preferred_element_type=jnp.float32)


def _rel_matmul(x, weight):
    m, k = x.shape
    grid = (NUM_RELS, m // MBLK)
    return pl.pallas_call(
        _rel_mm_body,
        grid=grid,
        in_specs=[
            pl.BlockSpec((MBLK, k), lambda r, i: (i, 0)),
            pl.BlockSpec((1, k, HID), lambda r, i: (r, 0, 0)),
        ],
        out_specs=pl.BlockSpec((1, MBLK, HID), lambda r, i: (r, i, 0)),
        out_shape=jax.ShapeDtypeStruct((NUM_RELS, m, HID), jnp.float32),
    )(x, weight)


def _out_body(x_ref, w_ref, b_ref, o_ref):
    logits = jnp.dot(x_ref[...], w_ref[...], preferred_element_type=jnp.float32)
    logits = logits + b_ref[...]
    mx = jnp.max(logits, axis=-1, keepdims=True)
    e = jnp.exp(logits - mx)
    o_ref[...] = e / jnp.sum(e, axis=-1, keepdims=True)


def _out_proj(x, W_out, b_out):
    m = x.shape[0]
    return pl.pallas_call(
        _out_body,
        grid=(m // MBLK,),
        in_specs=[
            pl.BlockSpec((MBLK, HID), lambda i: (i, 0)),
            pl.BlockSpec((HID, OUT_DIM), lambda i: (0, 0)),
            pl.BlockSpec((1, OUT_DIM), lambda i: (0, 0)),
        ],
        out_specs=pl.BlockSpec((MBLK, OUT_DIM), lambda i: (i, 0)),
        out_shape=jax.ShapeDtypeStruct((m, OUT_DIM), jnp.float32),
    )(x, W_out, b_out.reshape(1, OUT_DIM))


def kernel(g, h, r, norm, W_fnn, b_fnn, basis, coef, h_bias, W_out, b_out):
    weight = (coef @ basis.reshape(NUM_BASES, -1)).reshape(NUM_RELS, HID, HID)
    src = g[0]
    dst = g[1]
    rr = r.reshape(-1)
    nrm = norm.reshape(-1, 1)

    x = _fnn(h, W_fnn, b_fnn)
    x = jnp.concatenate([x, jnp.zeros_like(x)], axis=-1)

    for _ in range(2):
        Hr = _rel_matmul(x, weight)
        msg = Hr[rr, src] * nrm
        agg = jnp.zeros((N, HID), jnp.float32).at[dst].add(msg)
        x = jax.nn.relu(agg + h_bias)

    return _out_proj(x, W_out, b_out)
